# baseline (speedup 1.0000x reference)
import functools

import jax
import jax.numpy as jnp
from jax import lax
from jax.experimental import pallas as pl
from jax.experimental.pallas import tpu as pltpu
from jax.experimental.pallas import tpu_sc as plsc

_NC = 2
_NS = 16
_L = 16
_NBUF = 3


def _sc_scatter_partials(idx1, upd1, Mp):
    n = idx1.shape[0]
    NW = _NC * _NS
    B = 8192
    per_w = n // NW
    nblk = per_w // B
    ZB = 1024
    per_s = Mp // _NS

    mesh = plsc.VectorSubcoreMesh(
        core_axis_name="c", subcore_axis_name="s",
        num_cores=_NC, num_subcores=_NS)

    scratch = []
    for _ in range(_NBUF):
        scratch += [pltpu.VMEM((B,), jnp.int32),
                    pltpu.VMEM((B,), jnp.float32),
                    pltpu.SemaphoreType.DMA,
                    pltpu.SemaphoreType.DMA]
    scratch += [pltpu.VMEM((ZB,), jnp.float32),
                pltpu.VMEM_SHARED((Mp,), jnp.float32),
                pltpu.SemaphoreType.DMA]

    @functools.partial(
        pl.kernel,
        out_type=[jax.ShapeDtypeStruct((Mp,), jnp.float32),
                  jax.ShapeDtypeStruct((Mp,), jnp.float32)],
        mesh=mesh,
        scratch_types=scratch,
    )
    def k(idx_hbm, upd_hbm, out0, out1, *rest):
        bufs = tuple(rest[4 * i:4 * i + 4] for i in range(_NBUF))
        zero_v = rest[4 * _NBUF]
        acc = rest[4 * _NBUF + 1]
        sem_z = rest[4 * _NBUF + 2]
        c = lax.axis_index("c")
        s = lax.axis_index("s")
        w = c * _NS + s

        def stage(b):
            iv, uv, sem_st, _ = bufs[b % _NBUF]
            base = (w * nblk + b) * B
            ci = pltpu.async_copy(idx_hbm.at[pl.ds(base, B)], iv, sem_st)
            cu = pltpu.async_copy(upd_hbm.at[pl.ds(base, B)], uv, sem_st)
            return ci, cu

        pend = {0: stage(0)}
        if nblk > 1:
            pend[1] = stage(1)

        def zstore(i, carry):
            zero_v[pl.ds(i * _L, _L)] = jnp.zeros((_L,), jnp.float32)
            return carry
        lax.fori_loop(0, ZB // _L, zstore, 0)

        zcps = [pltpu.async_copy(
                    zero_v, acc.at[pl.ds(s * per_s + i * ZB, ZB)], sem_z)
                for i in range(per_s // ZB)]
        for cp in zcps:
            cp.wait()
        plsc.subcore_barrier()

        scat = {}
        for b in range(nblk):
            iv, uv, _, sem_sc = bufs[b % _NBUF]
            ci, cu = pend.pop(b)
            ci.wait()
            cu.wait()
            scat[b] = pltpu.async_copy(uv, acc.at[iv], sem_sc, add=True)
            if b + 2 < nblk:
                if b >= 1:
                    scat.pop(b - 1).wait()
                pend[b + 2] = stage(b + 2)
        for b in sorted(scat):
            scat.pop(b).wait()
        plsc.subcore_barrier()

        @pl.when(c == 0)
        def _():
            pltpu.sync_copy(acc.at[pl.ds(s * per_s, per_s)],
                            out0.at[pl.ds(s * per_s, per_s)])

        @pl.when(c == 1)
        def _():
            pltpu.sync_copy(acc.at[pl.ds(s * per_s, per_s)],
                            out1.at[pl.ds(s * per_s, per_s)])

    return k(idx1, upd1)


def _combine(d, a, b):
    M = d.shape[0]
    Mp = a.shape[0]
    BLK = Mp // 2

    def body(d_ref, a_ref, b_ref, o_ref):
        o_ref[...] = d_ref[...] + a_ref[...] + b_ref[...]

    return pl.pallas_call(
        body,
        grid=(Mp // BLK,),
        in_specs=[pl.BlockSpec((BLK,), lambda i: (i,))] * 3,
        out_specs=pl.BlockSpec((BLK,), lambda i: (i,)),
        out_shape=jax.ShapeDtypeStruct((M,), jnp.float32),
    )(d, a, b)


def kernel(data, indices, updates):
    Mp = 1 << 20
    q0, q1 = _sc_scatter_partials(indices.astype(jnp.int32), updates, Mp)
    return _combine(data, q0, q1)


# speedup vs baseline: 1.0030x; 1.0030x over previous
import functools

import jax
import jax.numpy as jnp
from jax import lax
from jax.experimental import pallas as pl
from jax.experimental.pallas import tpu as pltpu
from jax.experimental.pallas import tpu_sc as plsc

_NC = 2
_NS = 16
_L = 16
_NBUF = 3


def _sc_scatter_partials(idx1, upd1, Mp):
    n = idx1.shape[0]
    NW = _NC * _NS
    B = 8192
    per_w = n // NW
    nblk = per_w // B
    ZB = 2048
    per_s = Mp // _NS

    mesh = plsc.VectorSubcoreMesh(
        core_axis_name="c", subcore_axis_name="s",
        num_cores=_NC, num_subcores=_NS)

    scratch = []
    for _ in range(_NBUF):
        scratch += [pltpu.VMEM((B,), jnp.int32),
                    pltpu.VMEM((B,), jnp.float32),
                    pltpu.SemaphoreType.DMA,
                    pltpu.SemaphoreType.DMA]
    scratch += [pltpu.VMEM((ZB,), jnp.float32),
                pltpu.VMEM_SHARED((Mp,), jnp.float32),
                pltpu.SemaphoreType.DMA]

    @functools.partial(
        pl.kernel,
        out_type=[jax.ShapeDtypeStruct((Mp,), jnp.float32),
                  jax.ShapeDtypeStruct((Mp,), jnp.float32)],
        mesh=mesh,
        scratch_types=scratch,
    )
    def k(idx_hbm, upd_hbm, out0, out1, *rest):
        bufs = tuple(rest[4 * i:4 * i + 4] for i in range(_NBUF))
        zero_v = rest[4 * _NBUF]
        acc = rest[4 * _NBUF + 1]
        sem_z = rest[4 * _NBUF + 2]
        c = lax.axis_index("c")
        s = lax.axis_index("s")
        w = c * _NS + s

        def stage(b):
            iv, uv, sem_st, _ = bufs[b % _NBUF]
            base = (w * nblk + b) * B
            ci = pltpu.async_copy(idx_hbm.at[pl.ds(base, B)], iv, sem_st)
            cu = pltpu.async_copy(upd_hbm.at[pl.ds(base, B)], uv, sem_st)
            return ci, cu

        pend = {0: stage(0)}
        if nblk > 1:
            pend[1] = stage(1)

        def zstore(i, carry):
            zero_v[pl.ds(i * _L, _L)] = jnp.zeros((_L,), jnp.float32)
            return carry
        lax.fori_loop(0, ZB // _L, zstore, 0)

        zcps = [pltpu.async_copy(
                    zero_v, acc.at[pl.ds(s * per_s + i * ZB, ZB)], sem_z)
                for i in range(per_s // ZB)]
        for cp in zcps:
            cp.wait()
        plsc.subcore_barrier()

        scat = {}
        for b in range(nblk):
            iv, uv, _, sem_sc = bufs[b % _NBUF]
            ci, cu = pend.pop(b)
            ci.wait()
            cu.wait()
            scat[b] = pltpu.async_copy(uv, acc.at[iv], sem_sc, add=True)
            if b + 2 < nblk:
                if b >= 1:
                    scat.pop(b - 1).wait()
                pend[b + 2] = stage(b + 2)
        for b in sorted(scat):
            scat.pop(b).wait()
        plsc.subcore_barrier()

        @pl.when(c == 0)
        def _():
            pltpu.sync_copy(acc.at[pl.ds(s * per_s, per_s)],
                            out0.at[pl.ds(s * per_s, per_s)])

        @pl.when(c == 1)
        def _():
            pltpu.sync_copy(acc.at[pl.ds(s * per_s, per_s)],
                            out1.at[pl.ds(s * per_s, per_s)])

    return k(idx1, upd1)


def _combine(d, a, b):
    M = d.shape[0]
    Mp = a.shape[0]
    BLK = Mp // 2

    def body(d_ref, a_ref, b_ref, o_ref):
        o_ref[...] = d_ref[...] + a_ref[...] + b_ref[...]

    return pl.pallas_call(
        body,
        grid=(Mp // BLK,),
        in_specs=[pl.BlockSpec((BLK,), lambda i: (i,))] * 3,
        out_specs=pl.BlockSpec((BLK,), lambda i: (i,)),
        out_shape=jax.ShapeDtypeStruct((M,), jnp.float32),
    )(d, a, b)


def kernel(data, indices, updates):
    Mp = 1 << 20
    q0, q1 = _sc_scatter_partials(indices.astype(jnp.int32), updates, Mp)
    return _combine(data, q0, q1)
